# baseline (speedup 1.0000x reference)
import functools

import jax
import jax.numpy as jnp
from jax.experimental import pallas as pl

_B, _N, _H, _D, _K = 4, 4096, 256, 64, 32
_BLK = 512
_NCHUNK = _N // 128
_ONE_BITS = 0x3F800000


def _dot(a, b, dims):
    return jax.lax.dot_general(a, b, (dims, ((), ())),
                               preferred_element_type=jnp.float32)


def _m_kernel(h_ref, w1_ref, b1_ref, w2_ref, b2_ref, m1_ref, m2_ref):
    hm = h_ref[0]
    m1_ref[0] = jnp.tanh(_dot(hm, w1_ref[...], ((1,), (0,))) + b1_ref[...])
    m2_ref[0] = jnp.tanh(_dot(hm, w2_ref[...], ((1,), (0,))) + b2_ref[...])


def _main_kernel(m1_ref, m2_ref, selp_ref, o_ref):
    i = pl.program_id(1)
    m1a = m1_ref[0]
    m2a = m2_ref[0]
    m1r = m1_ref[0, pl.ds(i * _BLK, _BLK), :]
    m2r = m2_ref[0, pl.ds(i * _BLK, _BLK), :]
    s = _dot(m1r, m2a, ((1,), (1,))) - _dot(m2r, m1a, ((1,), (1,)))
    a = jax.nn.sigmoid(s)
    bits = jax.lax.bitcast_convert_type(a, jnp.int32)

    cmax = bits[:, 0:128]
    for j in range(1, _NCHUNK):
        cmax = jnp.maximum(cmax, bits[:, j * 128:(j + 1) * 128])
    rmax = jnp.max(cmax, axis=1, keepdims=True)

    def pre_cond(carry):
        lo, hi = carry
        return jnp.max(hi - lo) > 1

    def pre_body(carry):
        lo, hi = carry
        mid = lo + ((hi - lo) >> 1)
        cnt = jnp.sum((cmax >= mid).astype(jnp.int32), axis=1, keepdims=True)
        take = cnt >= _K
        return jnp.where(take, mid, lo), jnp.where(take, hi, mid)

    v, _ = jax.lax.while_loop(
        pre_cond, pre_body,
        (jnp.min(cmax, axis=1, keepdims=True), rmax + 1))

    kf = jnp.float32(_K)

    def val_cond(carry):
        lo, hi, _ = carry
        return jnp.max(hi - lo) > 1

    def val_body(carry):
        lo, hi, cnt_hi = carry
        mid = lo + ((hi - lo) >> 1)
        ge_f = (bits >= mid).astype(jnp.float32)
        cnt = _dot(ge_f, selp_ref[...], ((1,), (0,)))[:, _NCHUNK - 1:_NCHUNK]
        take = cnt >= kf
        return (jnp.where(take, mid, lo), jnp.where(take, hi, mid),
                jnp.where(take, cnt_hi, cnt))

    cnt_hi0 = jnp.zeros((_BLK, 1), jnp.float32)
    t, _, cnt_gt = jax.lax.while_loop(val_cond, val_body, (v, rmax + 1, cnt_hi0))

    gt = bits > t
    eq = bits == t
    r = kf - cnt_gt

    eq_f = eq.astype(jnp.float32)
    p = _dot(eq_f, selp_ref[...], ((1,), (0,)))
    p_incl = p[:, :_NCHUNK]
    p_excl = p[:, _NCHUNK:]
    j_iota = jax.lax.broadcasted_iota(jnp.int32, (_BLK, _NCHUNK), 1)
    c_star = jnp.min(jnp.where(p_incl >= r, j_iota, _NCHUNK), axis=1,
                     keepdims=True)
    pe_star = jnp.max(jnp.where(j_iota == c_star, p_excl, -1.0), axis=1,
                      keepdims=True)
    r2 = r - pe_star

    eq_star = jnp.zeros((_BLK, 128), jnp.bool_)
    for j in range(_NCHUNK):
        eq_star = eq_star | (eq[:, j * 128:(j + 1) * 128] & (c_star == j))
    lane = jax.lax.broadcasted_iota(jnp.int32, (_BLK, 128), 1)

    def lane_body(_, carry):
        lo, hi = carry
        mid = lo + ((hi - lo) >> 1)
        cnt = jnp.sum((eq_star & (lane <= mid)).astype(jnp.float32), axis=1,
                      keepdims=True)
        take = cnt >= r2
        return jnp.where(take, lo, mid), jnp.where(take, mid, hi)

    llo0 = jnp.full((_BLK, 1), -1, jnp.int32)
    lhi0 = jnp.full((_BLK, 1), 127, jnp.int32)
    _, l_star = jax.lax.fori_loop(0, 7, lane_body, (llo0, lhi0))
    m_col = c_star * 128 + l_star

    col = jax.lax.broadcasted_iota(jnp.int32, (_BLK, _N), 1)
    keep = gt | (eq & (col <= m_col))
    row_g = i * _BLK + jax.lax.broadcasted_iota(jnp.int32, (_BLK, _N), 0)
    o_ref[0] = a * keep.astype(jnp.float32) + (col == row_g).astype(jnp.float32)


def kernel(h_inv, W1, b1, W2, b2):
    m1, m2 = pl.pallas_call(
        _m_kernel,
        grid=(_B,),
        in_specs=[
            pl.BlockSpec((1, _N, _H), lambda b: (b, 0, 0)),
            pl.BlockSpec((_H, _D), lambda b: (0, 0)),
            pl.BlockSpec((1, _D), lambda b: (0, 0)),
            pl.BlockSpec((_H, _D), lambda b: (0, 0)),
            pl.BlockSpec((1, _D), lambda b: (0, 0)),
        ],
        out_specs=[
            pl.BlockSpec((1, _N, _D), lambda b: (b, 0, 0)),
            pl.BlockSpec((1, _N, _D), lambda b: (b, 0, 0)),
        ],
        out_shape=[
            jax.ShapeDtypeStruct((_B, _N, _D), jnp.float32),
            jax.ShapeDtypeStruct((_B, _N, _D), jnp.float32),
        ],
    )(h_inv, W1, b1.reshape(1, _D), W2, b2.reshape(1, _D))

    chunk = jnp.arange(_N, dtype=jnp.int32)[:, None] // 128
    jj = jnp.arange(_NCHUNK, dtype=jnp.int32)[None, :]
    selp = jnp.concatenate([(chunk <= jj), (chunk < jj)], axis=1
                           ).astype(jnp.float32)

    out = pl.pallas_call(
        _main_kernel,
        grid=(_B, _N // _BLK),
        in_specs=[
            pl.BlockSpec((1, _N, _D), lambda b, i: (b, 0, 0)),
            pl.BlockSpec((1, _N, _D), lambda b, i: (b, 0, 0)),
            pl.BlockSpec((_N, 2 * _NCHUNK), lambda b, i: (0, 0)),
        ],
        out_specs=pl.BlockSpec((1, _BLK, _N), lambda b, i: (b, i, 0)),
        out_shape=jax.ShapeDtypeStruct((_B, _N, _N), jnp.float32),
    )(m1, m2, selp)
    return out


# speedup vs baseline: 1.2781x; 1.2781x over previous
import functools

import jax
import jax.numpy as jnp
from jax.experimental import pallas as pl

_B, _N, _H, _D, _K = 4, 4096, 256, 64, 32
_BLK = 512
_NCHUNK = _N // 128
_ONE_BITS = 0x3F800000


def _dot(a, b, dims):
    return jax.lax.dot_general(a, b, (dims, ((), ())),
                               preferred_element_type=jnp.float32)


def _m_kernel(h_ref, w1_ref, b1_ref, w2_ref, b2_ref, m1_ref, m2_ref):
    hm = h_ref[0]
    m1_ref[0] = jnp.tanh(_dot(hm, w1_ref[...], ((1,), (0,))) + b1_ref[...])
    m2_ref[0] = jnp.tanh(_dot(hm, w2_ref[...], ((1,), (0,))) + b2_ref[...])


def _main_kernel(m1_ref, m2_ref, selp_ref, o_ref):
    i = pl.program_id(1)
    m1a = m1_ref[0]
    m2a = m2_ref[0]
    m1r = m1_ref[0, pl.ds(i * _BLK, _BLK), :]
    m2r = m2_ref[0, pl.ds(i * _BLK, _BLK), :]
    s = _dot(m1r, m2a, ((1,), (1,))) - _dot(m2r, m1a, ((1,), (1,)))
    a = jax.nn.sigmoid(s)
    bits = jax.lax.bitcast_convert_type(a, jnp.int32)

    cmax = bits[:, 0:128]
    for j in range(1, _NCHUNK):
        cmax = jnp.maximum(cmax, bits[:, j * 128:(j + 1) * 128])
    rmax = jnp.max(cmax, axis=1, keepdims=True)

    def pre_cond(carry):
        lo, hi = carry
        return jnp.max(hi - lo) > 1

    def pre_body(carry):
        lo, hi = carry
        mid = lo + ((hi - lo) >> 1)
        cnt = jnp.sum((cmax >= mid).astype(jnp.int32), axis=1, keepdims=True)
        take = cnt >= _K
        return jnp.where(take, mid, lo), jnp.where(take, hi, mid)

    v, _ = jax.lax.while_loop(
        pre_cond, pre_body,
        (jnp.min(cmax, axis=1, keepdims=True), rmax + 1))

    kf = jnp.float32(_K)

    def val_cond(carry):
        lo, hi, _ = carry
        return jnp.max(hi - lo) > 1

    def val_body(carry):
        lo, hi, cnt_hi = carry
        mid = lo + ((hi - lo) >> 1)
        cnt = jnp.sum((bits >= mid).astype(jnp.float32), axis=1, keepdims=True)
        take = cnt >= kf
        return (jnp.where(take, mid, lo), jnp.where(take, hi, mid),
                jnp.where(take, cnt_hi, cnt))

    cnt_hi0 = jnp.zeros((_BLK, 1), jnp.float32)
    t, _, cnt_gt = jax.lax.while_loop(val_cond, val_body, (v, rmax + 1, cnt_hi0))

    gt = bits > t
    eq = bits == t
    r = kf - cnt_gt

    eq_f = eq.astype(jnp.float32)
    p = _dot(eq_f, selp_ref[...], ((1,), (0,)))
    p_incl = p[:, :_NCHUNK]
    p_excl = p[:, _NCHUNK:]
    j_iota = jax.lax.broadcasted_iota(jnp.int32, (_BLK, _NCHUNK), 1)
    c_star = jnp.min(jnp.where(p_incl >= r, j_iota, _NCHUNK), axis=1,
                     keepdims=True)
    pe_star = jnp.max(jnp.where(j_iota == c_star, p_excl, -1.0), axis=1,
                      keepdims=True)
    r2 = r - pe_star

    eq_star = jnp.zeros((_BLK, 128), jnp.bool_)
    for j in range(_NCHUNK):
        eq_star = eq_star | (eq[:, j * 128:(j + 1) * 128] & (c_star == j))
    lane = jax.lax.broadcasted_iota(jnp.int32, (_BLK, 128), 1)

    def lane_body(_, carry):
        lo, hi = carry
        mid = lo + ((hi - lo) >> 1)
        cnt = jnp.sum((eq_star & (lane <= mid)).astype(jnp.float32), axis=1,
                      keepdims=True)
        take = cnt >= r2
        return jnp.where(take, lo, mid), jnp.where(take, mid, hi)

    llo0 = jnp.full((_BLK, 1), -1, jnp.int32)
    lhi0 = jnp.full((_BLK, 1), 127, jnp.int32)
    _, l_star = jax.lax.fori_loop(0, 7, lane_body, (llo0, lhi0))
    m_col = c_star * 128 + l_star

    col = jax.lax.broadcasted_iota(jnp.int32, (_BLK, _N), 1)
    keep = gt | (eq & (col <= m_col))
    row_g = i * _BLK + jax.lax.broadcasted_iota(jnp.int32, (_BLK, _N), 0)
    o_ref[0] = a * keep.astype(jnp.float32) + (col == row_g).astype(jnp.float32)


def kernel(h_inv, W1, b1, W2, b2):
    m1, m2 = pl.pallas_call(
        _m_kernel,
        grid=(_B,),
        in_specs=[
            pl.BlockSpec((1, _N, _H), lambda b: (b, 0, 0)),
            pl.BlockSpec((_H, _D), lambda b: (0, 0)),
            pl.BlockSpec((1, _D), lambda b: (0, 0)),
            pl.BlockSpec((_H, _D), lambda b: (0, 0)),
            pl.BlockSpec((1, _D), lambda b: (0, 0)),
        ],
        out_specs=[
            pl.BlockSpec((1, _N, _D), lambda b: (b, 0, 0)),
            pl.BlockSpec((1, _N, _D), lambda b: (b, 0, 0)),
        ],
        out_shape=[
            jax.ShapeDtypeStruct((_B, _N, _D), jnp.float32),
            jax.ShapeDtypeStruct((_B, _N, _D), jnp.float32),
        ],
    )(h_inv, W1, b1.reshape(1, _D), W2, b2.reshape(1, _D))

    chunk = jnp.arange(_N, dtype=jnp.int32)[:, None] // 128
    jj = jnp.arange(_NCHUNK, dtype=jnp.int32)[None, :]
    selp = jnp.concatenate([(chunk <= jj), (chunk < jj)], axis=1
                           ).astype(jnp.float32)

    out = pl.pallas_call(
        _main_kernel,
        grid=(_B, _N // _BLK),
        in_specs=[
            pl.BlockSpec((1, _N, _D), lambda b, i: (b, 0, 0)),
            pl.BlockSpec((1, _N, _D), lambda b, i: (b, 0, 0)),
            pl.BlockSpec((_N, 2 * _NCHUNK), lambda b, i: (0, 0)),
        ],
        out_specs=pl.BlockSpec((1, _BLK, _N), lambda b, i: (b, i, 0)),
        out_shape=jax.ShapeDtypeStruct((_B, _N, _N), jnp.float32),
    )(m1, m2, selp)
    return out
